# Initial kernel scaffold; baseline (speedup 1.0000x reference)
import jax
import jax.numpy as jnp
from jax.experimental import pallas as pl


def kernel(nodes, edge_pair, edge_attr, en_W1, en_b1, en_W2, en_b2, ee_W1, ee_b1, ee_W2, ee_b2, ge_W1, ge_b1, ge_W2, ge_b2, gn_W1, gn_b1, gn_W2, gn_b2, d_W1, d_b1, d_W2, d_b2):
    raise NotImplementedError("write your pallas kernel here")


# speedup vs baseline: 2.9905x; 2.9905x over previous
import functools

import jax
import jax.numpy as jnp
from jax import lax
from jax.experimental import pallas as pl
from jax.experimental.pallas import tpu as pltpu
from jax.experimental.pallas import tpu_sc as plsc

HIS = 10
ROLL = 2
MP = 2

NC = 2
NS = 16
NW = NC * NS
CHUNK = 80


def _sc_gather2(tab_a, tab_b, srcv, dstv):
    E = srcv.shape[0]
    W = tab_a.shape[1]
    ew = E // NW
    nch = ew // CHUNK
    assert ew % CHUNK == 0
    mesh = plsc.VectorSubcoreMesh(core_axis_name="c", subcore_axis_name="s")

    @functools.partial(
        pl.kernel,
        out_type=[jax.ShapeDtypeStruct((E, W), jnp.float32),
                  jax.ShapeDtypeStruct((E, W), jnp.float32)],
        mesh=mesh,
        compiler_params=pltpu.CompilerParams(use_tc_tiling_on_sc=False),
        scratch_types=[
            pltpu.VMEM((CHUNK,), jnp.int32),
            pltpu.VMEM((CHUNK,), jnp.int32),
            pltpu.VMEM((CHUNK, W), jnp.float32),
            pltpu.VMEM((CHUNK, W), jnp.float32),
            pltpu.SemaphoreType.DMA,
            pltpu.SemaphoreType.DMA,
        ],
    )
    def k(a_h, b_h, s_h, d_h, ga_h, gb_h, si, di, ra, rb, sa, sb):
        wid = lax.axis_index("s") * NC + lax.axis_index("c")
        base = wid * ew

        def body(i, carry):
            off = base + i * CHUNK
            pltpu.sync_copy(s_h.at[pl.ds(off, CHUNK)], si)
            pltpu.sync_copy(d_h.at[pl.ds(off, CHUNK)], di)
            cpa = pltpu.async_copy(a_h.at[si], ra, sa)
            cpb = pltpu.async_copy(b_h.at[di], rb, sb)
            cpa.wait()
            cpb.wait()
            pltpu.sync_copy(ra, ga_h.at[pl.ds(off, CHUNK)])
            pltpu.sync_copy(rb, gb_h.at[pl.ds(off, CHUNK)])
            return carry

        lax.fori_loop(0, nch, body, 0)

    return k(tab_a, tab_b, srcv, dstv)


def _sc_scatter_add(f, srcv, dstv, n_nodes, zeros_tab):
    E, Wf = f.shape
    ew = E // NW
    nch = ew // CHUNK
    assert ew % CHUNK == 0
    rows = n_nodes // NS
    mesh = plsc.VectorSubcoreMesh(core_axis_name="c", subcore_axis_name="s")

    @functools.partial(
        pl.kernel,
        out_type=jax.ShapeDtypeStruct((2, n_nodes, Wf), jnp.float32),
        mesh=mesh,
        compiler_params=pltpu.CompilerParams(use_tc_tiling_on_sc=False),
        scratch_types=[
            pltpu.VMEM((CHUNK,), jnp.int32),
            pltpu.VMEM((CHUNK,), jnp.int32),
            pltpu.VMEM((CHUNK, Wf), jnp.float32),
            pltpu.VMEM_SHARED((n_nodes, Wf), jnp.float32),
        ],
    )
    def k(f_h, s_h, d_h, z_h, out_h, si, di, fb, shared):
        cid = lax.axis_index("c")
        sid = lax.axis_index("s")
        pltpu.sync_copy(z_h.at[pl.ds(sid * rows, rows)],
                        shared.at[pl.ds(sid * rows, rows)])
        plsc.subcore_barrier()
        base = cid * (E // 2) + sid * ew

        def body(i, carry):
            off = base + i * CHUNK
            pltpu.sync_copy(s_h.at[pl.ds(off, CHUNK)], si)
            pltpu.sync_copy(d_h.at[pl.ds(off, CHUNK)], di)
            pltpu.sync_copy(f_h.at[pl.ds(off, CHUNK)], fb)
            pltpu.sync_copy(fb, shared.at[si], add=True)
            pltpu.sync_copy(fb, shared.at[di], add=True)
            return carry

        lax.fori_loop(0, nch, body, 0)
        plsc.subcore_barrier()
        pltpu.sync_copy(shared.at[pl.ds(sid * rows, rows)],
                        out_h.at[cid, pl.ds(sid * rows, rows)])

    return k(f, srcv, dstv, zeros_tab)


def _sc_count(srcv, dstv, n_nodes, zeros_tab, ones_chunk):
    E = srcv.shape[0]
    ew = E // NW
    nch = ew // CHUNK
    rows = n_nodes // NS
    mesh = plsc.VectorSubcoreMesh(core_axis_name="c", subcore_axis_name="s")

    @functools.partial(
        pl.kernel,
        out_type=jax.ShapeDtypeStruct((2, n_nodes, 16), jnp.float32),
        mesh=mesh,
        compiler_params=pltpu.CompilerParams(use_tc_tiling_on_sc=False),
        scratch_types=[
            pltpu.VMEM((CHUNK,), jnp.int32),
            pltpu.VMEM((CHUNK,), jnp.int32),
            pltpu.VMEM((CHUNK, 16), jnp.float32),
            pltpu.VMEM_SHARED((n_nodes, 16), jnp.float32),
        ],
    )
    def k(s_h, d_h, z_h, o_h, out_h, si, di, ob, shared):
        cid = lax.axis_index("c")
        sid = lax.axis_index("s")
        pltpu.sync_copy(z_h.at[pl.ds(sid * rows, rows)],
                        shared.at[pl.ds(sid * rows, rows)])
        pltpu.sync_copy(o_h, ob)
        plsc.subcore_barrier()
        base = cid * (E // 2) + sid * ew

        def body(i, carry):
            off = base + i * CHUNK
            pltpu.sync_copy(s_h.at[pl.ds(off, CHUNK)], si)
            pltpu.sync_copy(d_h.at[pl.ds(off, CHUNK)], di)
            pltpu.sync_copy(ob, shared.at[si], add=True)
            pltpu.sync_copy(ob, shared.at[di], add=True)
            return carry

        lax.fori_loop(0, nch, body, 0)
        plsc.subcore_barrier()
        pltpu.sync_copy(shared.at[pl.ds(sid * rows, rows)],
                        out_h.at[cid, pl.ds(sid * rows, rows)])

    return k(srcv, dstv, zeros_tab, ones_chunk)


def _elu(x):
    return jnp.where(x > 0, x, jnp.exp(jnp.minimum(x, 0.0)) - 1.0)


def _stats_from_sums(sums_pad, n_rows, width):
    s1 = sums_pad[0, :width]
    s2 = sums_pad[1, :width]
    mu = s1 / n_rows
    var = s2 / n_rows - mu * mu
    sg = jnp.sqrt(var + 1e-5)
    return mu, sg


def _pack_sums(f):
    s1 = jnp.sum(f, axis=0)
    s2 = jnp.sum(f * f, axis=0)
    w = f.shape[1]
    z = jnp.zeros((128 - w,), jnp.float32)
    row1 = jnp.concatenate([s1, z])[None]
    row2 = jnp.concatenate([s2, z])[None]
    return jnp.concatenate([row1, row2, jnp.zeros((6, 128), jnp.float32)],
                           axis=0)


def _tc_ee(edge_attr, gs, gd, ps, pd, ee_W1, ee_b1, ee_W2, ee_b2, first):
    E = edge_attr.shape[0]
    BE = 2000
    grid = E // BE

    def body(ea_r, gs_r, gd_r, *rest):
        if first:
            (w1_r, b1_r, w2_r, b2_r, out_r, sums_r) = rest
        else:
            (ps_r, pd_r, w1_r, b1_r, w2_r, b2_r, out_r, sums_r) = rest
        i = pl.program_id(0)
        d0 = jnp.abs(gs_r[...] - gd_r[...])
        if first:
            ea = jnp.concatenate([ea_r[...], d0[:, :6]], axis=1)
        else:
            dp = jnp.abs(ps_r[...] - pd_r[...])
            ea = jnp.concatenate([ea_r[...], d0[:, :2], dp[:, :4]], axis=1)
        h = _elu(jnp.dot(ea, w1_r[...], preferred_element_type=jnp.float32) + b1_r[...])
        f = _elu(jnp.dot(h, w2_r[...], preferred_element_type=jnp.float32) + b2_r[...])
        out_r[...] = f

        @pl.when(i == 0)
        def _():
            sums_r[...] = jnp.zeros((8, 128), jnp.float32)

        sums_r[...] += _pack_sums(f)

    full = lambda s: pl.BlockSpec(s, lambda i: tuple(0 for _ in s))
    pos_specs = [] if first else [pl.BlockSpec((BE, 16), lambda i: (i, 0))] * 2
    pos_args = [] if first else [ps, pd]
    return pl.pallas_call(
        body,
        grid=(grid,),
        in_specs=[
            pl.BlockSpec((BE, 2), lambda i: (i, 0)),
            pl.BlockSpec((BE, 16), lambda i: (i, 0)),
            pl.BlockSpec((BE, 16), lambda i: (i, 0)),
            *pos_specs,
            full((8, 32)),
            full((32,)),
            full((32, 32)),
            full((32,)),
        ],
        out_specs=[
            pl.BlockSpec((BE, 32), lambda i: (i, 0)),
            pl.BlockSpec((8, 128), lambda i: (0, 0)),
        ],
        out_shape=[
            jax.ShapeDtypeStruct((E, 32), jnp.float32),
            jax.ShapeDtypeStruct((8, 128), jnp.float32),
        ],
    )(edge_attr, gs, gd, *pos_args,
      jnp.pad(ee_W1, ((0, 2), (0, 0))), ee_b1, ee_W2, ee_b2)


def _tc_edge(ga, gb, le_raw, le_sums, f_prev, fp_sums, We, ge_b1, ge_W2, ge_b2,
             have_prev):
    E = ga.shape[0]
    BE = 2000
    grid = E // BE

    def body(ga_r, gb_r, le_r, ls_r, *rest):
        if have_prev:
            (fp_r, fs_r, we_r, b1_r, w2_r, b2_r, out_r, sums_r) = rest
        else:
            (we_r, b1_r, w2_r, b2_r, out_r, sums_r) = rest
        i = pl.program_id(0)
        mule, sgle = _stats_from_sums(ls_r[...], float(E), 32)
        le = (le_r[...] - mule) / sgle
        if have_prev:
            mup, sgp = _stats_from_sums(fs_r[...], float(E), 32)
            le = le + (fp_r[...] - mup) / sgp
        ce = jnp.dot(le, we_r[...], preferred_element_type=jnp.float32) + b1_r[...]
        h = _elu(ga_r[...] + gb_r[...] + ce)
        f = _elu(jnp.dot(h, w2_r[...], preferred_element_type=jnp.float32) + b2_r[...])
        out_r[...] = f

        @pl.when(i == 0)
        def _():
            sums_r[...] = jnp.zeros((8, 128), jnp.float32)

        sums_r[...] += _pack_sums(f)

    full = lambda s: pl.BlockSpec(s, lambda i: tuple(0 for _ in s))
    prev_specs = ([pl.BlockSpec((BE, 32), lambda i: (i, 0)), full((8, 128))]
                  if have_prev else [])
    prev_args = [f_prev, fp_sums] if have_prev else []
    return pl.pallas_call(
        body,
        grid=(grid,),
        in_specs=[
            pl.BlockSpec((BE, 128), lambda i: (i, 0)),
            pl.BlockSpec((BE, 128), lambda i: (i, 0)),
            pl.BlockSpec((BE, 32), lambda i: (i, 0)),
            full((8, 128)),
            *prev_specs,
            full((32, 128)),
            full((128,)),
            full((128, 32)),
            full((32,)),
        ],
        out_specs=[
            pl.BlockSpec((BE, 32), lambda i: (i, 0)),
            pl.BlockSpec((8, 128), lambda i: (0, 0)),
        ],
        out_shape=[
            jax.ShapeDtypeStruct((E, 32), jnp.float32),
            jax.ShapeDtypeStruct((8, 128), jnp.float32),
        ],
    )(ga, gb, le_raw, le_sums, *prev_args, We, ge_b1, ge_W2, ge_b2)


def _tc_node_front(hist, en_W1, en_b1, en_W2, en_b2, Ws, Wd):
    N = hist.shape[0]

    def body(h_r, w1_r, b1_r, w2_r, b2_r, ws_r, wd_r, ln_r, a_r, b_r):
        h = _elu(jnp.dot(h_r[...], w1_r[...], preferred_element_type=jnp.float32) + b1_r[...])
        f = _elu(jnp.dot(h, w2_r[...], preferred_element_type=jnp.float32) + b2_r[...])
        mu = jnp.mean(f, axis=0)
        var = jnp.mean(f * f, axis=0) - mu * mu
        ln = (f - mu) / jnp.sqrt(var + 1e-5)
        ln_r[...] = ln
        a_r[...] = jnp.dot(ln, ws_r[...], preferred_element_type=jnp.float32)
        b_r[...] = jnp.dot(ln, wd_r[...], preferred_element_type=jnp.float32)

    return pl.pallas_call(
        body,
        out_shape=[jax.ShapeDtypeStruct((N, 128), jnp.float32)] * 3,
    )(hist, en_W1, en_b1, en_W2, en_b2, Ws, Wd)


def _tc_node(ln, parts, cntp, f_sums, n_edges, gn_W1a, gn_W1b, gn_b1, gn_W2,
             gn_b2, Ws, Wd, d_W1, d_b1, d_W2p, d_b2p, last):
    N = ln.shape[0]

    if last:
        def body_last(ln_r, pa_r, cp_r, fs_r, w1a_r, w1b_r, b1_r, w2_r, b2_r,
                      dw1_r, db1_r, dw2_r, db2_r, acc_r):
            mu, sg = _stats_from_sums(fs_r[...], float(n_edges), 32)
            cnt = (cp_r[0, :, 0] + cp_r[1, :, 0])[:, None]
            rawagg = pa_r[0] + pa_r[1]
            agg = (rawagg - cnt * mu) / sg
            h = _elu(jnp.dot(ln_r[...], w1a_r[...], preferred_element_type=jnp.float32)
                     + jnp.dot(agg, w1b_r[...], preferred_element_type=jnp.float32)
                     + b1_r[...])
            g = _elu(jnp.dot(h, w2_r[...], preferred_element_type=jnp.float32) + b2_r[...])
            gmu = jnp.mean(g, axis=0)
            gvar = jnp.mean(g * g, axis=0) - gmu * gmu
            ln2 = (g - gmu) / jnp.sqrt(gvar + 1e-5) + ln_r[...]
            t = jnp.tanh(jnp.dot(ln2, dw1_r[...], preferred_element_type=jnp.float32) + db1_r[...])
            acc_r[...] = jnp.dot(t, dw2_r[...], preferred_element_type=jnp.float32) + db2_r[...]

        return pl.pallas_call(
            body_last,
            out_shape=jax.ShapeDtypeStruct((N, 8), jnp.float32),
        )(ln, parts, cntp, f_sums, gn_W1a, gn_W1b, gn_b1, gn_W2, gn_b2,
          d_W1, d_b1, d_W2p, d_b2p)

    def body_mid(ln_r, pa_r, cp_r, fs_r, w1a_r, w1b_r, b1_r, w2_r, b2_r,
                 ws_r, wd_r, ln2_r, a_r, b_r):
        mu, sg = _stats_from_sums(fs_r[...], float(n_edges), 32)
        cnt = (cp_r[0, :, 0] + cp_r[1, :, 0])[:, None]
        rawagg = pa_r[0] + pa_r[1]
        agg = (rawagg - cnt * mu) / sg
        h = _elu(jnp.dot(ln_r[...], w1a_r[...], preferred_element_type=jnp.float32)
                 + jnp.dot(agg, w1b_r[...], preferred_element_type=jnp.float32)
                 + b1_r[...])
        g = _elu(jnp.dot(h, w2_r[...], preferred_element_type=jnp.float32) + b2_r[...])
        gmu = jnp.mean(g, axis=0)
        gvar = jnp.mean(g * g, axis=0) - gmu * gmu
        ln2 = (g - gmu) / jnp.sqrt(gvar + 1e-5) + ln_r[...]
        ln2_r[...] = ln2
        a_r[...] = jnp.dot(ln2, ws_r[...], preferred_element_type=jnp.float32)
        b_r[...] = jnp.dot(ln2, wd_r[...], preferred_element_type=jnp.float32)

    return pl.pallas_call(
        body_mid,
        out_shape=[jax.ShapeDtypeStruct((N, 128), jnp.float32)] * 3,
    )(ln, parts, cntp, f_sums, gn_W1a, gn_W1b, gn_b1, gn_W2, gn_b2, Ws, Wd)


def kernel(nodes, edge_pair, edge_attr, en_W1, en_b1, en_W2, en_b2, ee_W1,
           ee_b1, ee_W2, ee_b2, ge_W1, ge_b1, ge_W2, ge_b2, gn_W1, gn_b1,
           gn_W2, gn_b2, d_W1, d_b1, d_W2, d_b2):
    window = nodes[0]
    N = window.shape[1]
    E = edge_pair.shape[0]
    src = edge_pair[:, 0]
    dst = edge_pair[:, 1]
    gt_future = window[HIS:]
    n10 = window[HIS]
    hist = jnp.transpose(window[:HIS], (1, 0, 2)).reshape(N, HIS * 4)

    Ws, Wd, We = ge_W1[:128], ge_W1[128:256], ge_W1[256:]
    Gn1a, Gn1b = gn_W1[:128], gn_W1[128:]
    d_W2p = jnp.pad(d_W2, ((0, 0), (0, 6)))
    d_b2p = jnp.pad(d_b2, (0, 6))

    zeros32 = jnp.zeros((N, 32), jnp.float32)
    zeros16 = jnp.zeros((N, 16), jnp.float32)
    ones16 = jnp.ones((CHUNK, 16), jnp.float32)
    cntp = _sc_count(src, dst, N, zeros16, ones16)

    n10p = jnp.pad(n10, ((0, 0), (0, 12)))
    n10s, n10d = _sc_gather2(n10p, n10p, src, dst)

    cur_pos = window[HIS - 1, :, :2]
    cur_vel = window[HIS - 1, :, 2:]
    preds = []
    pos_s = pos_d = None
    for r in range(ROLL):
        ln, A, B = _tc_node_front(hist, en_W1, en_b1, en_W2, en_b2, Ws, Wd)
        le_raw, le_sums = _tc_ee(edge_attr, n10s, n10d, pos_s, pos_d,
                                 ee_W1, ee_b1, ee_W2, ee_b2, r == 0)
        f_prev, fp_sums = None, None
        for m in range(MP):
            ga, gb = _sc_gather2(A, B, src, dst)
            f, f_sums = _tc_edge(ga, gb, le_raw, le_sums, f_prev, fp_sums,
                                 We, ge_b1, ge_W2, ge_b2, m > 0)
            parts = _sc_scatter_add(f, src, dst, N, zeros32)
            last = m == MP - 1
            if last:
                acc8 = _tc_node(ln, parts, cntp, f_sums, E, Gn1a, Gn1b, gn_b1,
                                gn_W2, gn_b2, Ws, Wd, d_W1, d_b1, d_W2p,
                                d_b2p, True)
            else:
                ln, A, B = _tc_node(ln, parts, cntp, f_sums, E, Gn1a, Gn1b,
                                    gn_b1, gn_W2, gn_b2, Ws, Wd, d_W1, d_b1,
                                    d_W2p, d_b2p, False)
                f_prev, fp_sums = f, f_sums
        acc = acc8[:, :2]
        prev_pos = cur_pos
        cur_pos = 2 * cur_pos + acc - prev_pos
        cur_vel = cur_vel + acc
        preds.append(cur_pos)
        if r + 1 < ROLL:
            posp = jnp.pad(cur_pos, ((0, 0), (0, 14)))
            pos_s, pos_d = _sc_gather2(posp, posp, src, dst)
            hist = jnp.concatenate(
                [hist.reshape(N, HIS, 4)[:, 1:].reshape(N, (HIS - 1) * 4),
                 cur_pos, cur_vel], axis=-1)

    preds = jnp.stack(preds, axis=0)[None]
    target = gt_future[:, :, :2]
    diff = preds[0] - target
    loss_nll = (diff ** 2 / (2 * 5e-05)).sum() / (target.shape[0] * target.shape[1])
    loss_mse = (diff ** 2).mean()
    return (preds, loss_nll, loss_mse, gt_future)
